# Initial kernel scaffold; baseline (speedup 1.0000x reference)
import jax
import jax.numpy as jnp
from jax.experimental import pallas as pl


def kernel(confidences, hits, labels):
    raise NotImplementedError("write your pallas kernel here")


# speedup vs baseline: 187.8524x; 187.8524x over previous
import functools

import jax
import jax.numpy as jnp
from jax import lax
from jax.experimental import pallas as pl
from jax.experimental.pallas import tpu as pltpu
from jax.experimental.pallas import tpu_sc as plsc

N = 1_000_000
N_CLASS = 100
N_BINS = 15
PAD_C = 128
NW = 32
CHUNK = 31_264
N_PAD = NW * CHUNK
STEPS = CHUNK // 16
CELLS = 16 * PAD_C

_mesh = plsc.VectorSubcoreMesh(core_axis_name="c", subcore_axis_name="s")
_sc_params = pltpu.CompilerParams(needs_layout_passes=False)


def _wid():
    return lax.axis_index("c") * 16 + lax.axis_index("s")


def _div15(cnt):
    m0 = (cnt.astype(jnp.float32) * jnp.float32(1.0 / 15.0)).astype(jnp.int32)
    return m0 + (cnt >= (m0 + 1) * 15).astype(jnp.int32) \
              - (cnt < m0 * 15).astype(jnp.int32)


@functools.partial(
    pl.kernel,
    out_type=jax.ShapeDtypeStruct((NW, PAD_C), jnp.int32),
    mesh=_mesh,
    compiler_params=_sc_params,
    scratch_types=[pltpu.VMEM((CHUNK,), jnp.int32), pltpu.VMEM((PAD_C,), jnp.int32)],
)
def _hist_kernel(lab_hbm, hist_out, lab_v, h_v):
    w = _wid()
    pltpu.sync_copy(lab_hbm.at[pl.ds(w * CHUNK, CHUNK)], lab_v)
    zi = jnp.zeros((16,), jnp.int32)

    def zero(i, _):
        h_v[pl.ds(i * 16, 16)] = zi
        return 0

    lax.fori_loop(0, PAD_C // 16, zero, 0)
    ones = jnp.ones((16,), jnp.int32)

    def body(i, _):
        lab = lab_v[pl.ds(i * 16, 16)]
        plsc.addupdate_scatter(h_v, [lab], ones)
        return 0

    lax.fori_loop(0, STEPS, body, 0)
    pltpu.sync_copy(h_v, hist_out.at[w])


@functools.partial(
    pl.kernel,
    out_type=(jax.ShapeDtypeStruct((NW, CELLS), jnp.float32),
              jax.ShapeDtypeStruct((NW, CELLS), jnp.float32)),
    mesh=_mesh,
    compiler_params=_sc_params,
    scratch_types=[
        pltpu.VMEM((CHUNK,), jnp.int32),
        pltpu.VMEM((CHUNK,), jnp.float32),
        pltpu.VMEM((CHUNK,), jnp.float32),
        pltpu.VMEM((NW, PAD_C), jnp.int32),
        pltpu.VMEM((PAD_C,), jnp.int32),
        pltpu.VMEM((PAD_C,), jnp.int32),
        pltpu.VMEM((PAD_C,), jnp.float32),
        pltpu.VMEM((CELLS,), jnp.float32),
        pltpu.VMEM((CELLS,), jnp.float32),
        pltpu.SemaphoreType.DMA,
        pltpu.SemaphoreType.DMA,
        pltpu.SemaphoreType.DMA,
    ],
)
def _main_kernel(cf_hbm, ht_hbm, lab_hbm, hist_hbm, accp_out, acch_out,
                 lab_v, cf_v, ht_v, hist_v, cnt_v, mass_v, rcp_v,
                 accp_v, acch_v, sem1, sem2, sem3):
    w = _wid()
    base = w * CHUNK
    cp1 = pltpu.async_copy(lab_hbm.at[pl.ds(base, CHUNK)], lab_v, sem1)
    cp2 = pltpu.async_copy(cf_hbm.at[pl.ds(base, CHUNK)], cf_v, sem2)
    cp3 = pltpu.async_copy(ht_hbm.at[pl.ds(base, CHUNK)], ht_v, sem3)
    pltpu.sync_copy(hist_hbm, hist_v)

    zi = jnp.zeros((16,), jnp.int32)
    for j in range(PAD_C // 16):
        def acc_w(v, carry):
            tot, off = carry
            hv = hist_v[v, pl.ds(j * 16, 16)]
            sel = (v < w).astype(jnp.int32)
            return (tot + hv, off + hv * sel)

        tot, off = lax.fori_loop(0, NW, acc_w, (zi, zi))
        m = _div15(tot)
        mass_v[pl.ds(j * 16, 16)] = m
        rcp_v[pl.ds(j * 16, 16)] = jnp.float32(1.0) / m.astype(jnp.float32)
        cnt_v[pl.ds(j * 16, 16)] = off

    zf = jnp.zeros((16,), jnp.float32)

    def zero(i, _):
        accp_v[pl.ds(i * 16, 16)] = zf
        acch_v[pl.ds(i * 16, 16)] = zf
        return 0

    lax.fori_loop(0, CELLS // 16, zero, 0)
    cp1.wait()
    cp2.wait()
    cp3.wait()

    ones = jnp.ones((16,), jnp.int32)
    tmask = jnp.ones((16,), jnp.bool_)

    def body(i, _):
        o = i * 16
        lab = lab_v[pl.ds(o, 16)]
        cf = cf_v[pl.ds(o, 16)]
        ht = ht_v[pl.ds(o, 16)]
        cbase = plsc.load_gather(cnt_v, [lab])
        occ, _ = plsc.scan_count(lab, mask=tmask)
        rank = cbase + occ.astype(jnp.int32) - 1
        plsc.addupdate_scatter(cnt_v, [lab], ones)
        m = plsc.load_gather(mass_v, [lab])
        r = plsc.load_gather(rcp_v, [lab])
        q0 = (rank.astype(jnp.float32) * r).astype(jnp.int32)
        q0 = jnp.clip(q0, 0, 16)
        q = q0 + (rank >= (q0 + 1) * m).astype(jnp.int32) \
               - (rank < q0 * m).astype(jnp.int32)
        q = jnp.clip(q, 0, 15)
        idx = q * PAD_C + lab
        plsc.addupdate_scatter(accp_v, [idx], cf)
        plsc.addupdate_scatter(acch_v, [idx], ht)
        return 0

    lax.fori_loop(0, STEPS, body, 0)
    pltpu.sync_copy(accp_v, accp_out.at[w])
    pltpu.sync_copy(acch_v, acch_out.at[w])


def _final_body(accp_ref, acch_ref, hist_ref, out_ref):
    p = accp_ref[0]
    h = acch_ref[0]
    cnt = hist_ref[0]
    for v in range(1, NW):
        p = p + accp_ref[v]
        h = h + acch_ref[v]
        cnt = cnt + hist_ref[v]
    mass = _div15(cnt).astype(jnp.float32)
    coef = jnp.float32(1.0) / (mass * jnp.float32(N_BINS * N_CLASS))
    cls = lax.broadcasted_iota(jnp.int32, (1, PAD_C), 1)
    coef = jnp.where(cls < N_CLASS, coef, jnp.float32(0.0))
    d = jnp.abs(p - h) * coef
    b = lax.broadcasted_iota(jnp.int32, (16, PAD_C), 0)
    d = jnp.where(b < N_BINS, d, jnp.float32(0.0))
    out_ref[...] = jnp.sum(d).reshape(1, 1)


def kernel(confidences, hits, labels):
    pad = N_PAD - N
    cf = jnp.concatenate([confidences, jnp.zeros((pad,), jnp.float32)])
    ht = jnp.concatenate([hits, jnp.zeros((pad,), jnp.float32)])
    lab = jnp.concatenate([labels, jnp.full((pad,), PAD_C - 1, jnp.int32)])
    hist = _hist_kernel(lab)
    accp, acch = _main_kernel(cf, ht, lab, hist)
    out = pl.pallas_call(
        _final_body,
        out_shape=jax.ShapeDtypeStruct((1, 1), jnp.float32),
    )(accp.reshape(NW, 16, PAD_C), acch.reshape(NW, 16, PAD_C),
      hist.reshape(NW, 1, PAD_C))
    return out[0, 0]


# speedup vs baseline: 192.1901x; 1.0231x over previous
import functools

import jax
import jax.numpy as jnp
from jax import lax
from jax.experimental import pallas as pl
from jax.experimental.pallas import tpu as pltpu
from jax.experimental.pallas import tpu_sc as plsc

N = 1_000_000
N_CLASS = 100
N_BINS = 15
PAD_C = 128
NW = 32
NCH = 64
SCHUNK = 15_632
CHUNK = 2 * SCHUNK
N_PAD = NCH * SCHUNK
STEPS = SCHUNK // 16
CELLS = 16 * PAD_C

_mesh = plsc.VectorSubcoreMesh(core_axis_name="c", subcore_axis_name="s")
_sc_params = pltpu.CompilerParams(needs_layout_passes=False)


def _wid():
    return lax.axis_index("c") * 16 + lax.axis_index("s")


def _div15(cnt):
    m0 = (cnt.astype(jnp.float32) * jnp.float32(1.0 / 15.0)).astype(jnp.int32)
    return m0 + (cnt >= (m0 + 1) * 15).astype(jnp.int32) \
              - (cnt < m0 * 15).astype(jnp.int32)


@functools.partial(
    pl.kernel,
    out_type=jax.ShapeDtypeStruct((NCH * PAD_C,), jnp.int32),
    mesh=_mesh,
    compiler_params=_sc_params,
    scratch_types=[pltpu.VMEM((CHUNK,), jnp.int32),
                   pltpu.VMEM((2 * PAD_C,), jnp.int32)],
)
def _hist_kernel(lab_hbm, hist_out, lab_v, h_v):
    w = _wid()
    pltpu.sync_copy(lab_hbm.at[pl.ds(w * CHUNK, CHUNK)], lab_v)
    zi = jnp.zeros((16,), jnp.int32)

    def zero(i, _):
        h_v[pl.ds(i * 16, 16)] = zi
        return 0

    lax.fori_loop(0, 2 * PAD_C // 16, zero, 0)
    ones = jnp.ones((16,), jnp.int32)
    c128 = jnp.full((16,), PAD_C, jnp.int32)

    def body(i, _):
        o = i * 16
        lab_a = lab_v[pl.ds(o, 16)]
        lab_b = lab_v[pl.ds(SCHUNK + o, 16)]
        plsc.addupdate_scatter(h_v, [lab_a], ones)
        plsc.addupdate_scatter(h_v, [lab_b + c128], ones)
        return 0

    lax.fori_loop(0, STEPS, body, 0)
    pltpu.sync_copy(h_v, hist_out.at[pl.ds(w * 2 * PAD_C, 2 * PAD_C)])


@functools.partial(
    pl.kernel,
    out_type=(jax.ShapeDtypeStruct((NW, CELLS), jnp.float32),
              jax.ShapeDtypeStruct((NW, CELLS), jnp.float32)),
    mesh=_mesh,
    compiler_params=_sc_params,
    scratch_types=[
        pltpu.VMEM((CHUNK,), jnp.int32),
        pltpu.VMEM((CHUNK,), jnp.float32),
        pltpu.VMEM((CHUNK,), jnp.float32),
        pltpu.VMEM((NCH * PAD_C,), jnp.int32),
        pltpu.VMEM((PAD_C,), jnp.int32),
        pltpu.VMEM((PAD_C,), jnp.int32),
        pltpu.VMEM((PAD_C,), jnp.int32),
        pltpu.VMEM((PAD_C,), jnp.float32),
        pltpu.VMEM((CELLS,), jnp.float32),
        pltpu.VMEM((CELLS,), jnp.float32),
        pltpu.SemaphoreType.DMA,
        pltpu.SemaphoreType.DMA,
        pltpu.SemaphoreType.DMA,
    ],
)
def _main_kernel(cf_hbm, ht_hbm, lab_hbm, hist_hbm, accp_out, acch_out,
                 lab_v, cf_v, ht_v, hist_v, cnta_v, cntb_v, mass_v, rcp_v,
                 accp_v, acch_v, sem1, sem2, sem3):
    w = _wid()
    base = w * CHUNK
    cp1 = pltpu.async_copy(lab_hbm.at[pl.ds(base, CHUNK)], lab_v, sem1)
    cp2 = pltpu.async_copy(cf_hbm.at[pl.ds(base, CHUNK)], cf_v, sem2)
    cp3 = pltpu.async_copy(ht_hbm.at[pl.ds(base, CHUNK)], ht_v, sem3)
    pltpu.sync_copy(hist_hbm, hist_v)

    zi = jnp.zeros((16,), jnp.int32)
    ca = 2 * w
    for j in range(PAD_C // 16):
        def acc_v(v, carry):
            tot, off = carry
            hv = hist_v[pl.ds(v * PAD_C + j * 16, 16)]
            sel = (v < ca).astype(jnp.int32)
            return (tot + hv, off + hv * sel)

        tot, offa = lax.fori_loop(0, NCH, acc_v, (zi, zi))
        offb = offa + hist_v[pl.ds(ca * PAD_C + j * 16, 16)]
        m = _div15(tot)
        mass_v[pl.ds(j * 16, 16)] = m
        rcp_v[pl.ds(j * 16, 16)] = jnp.float32(1.0) / m.astype(jnp.float32)
        cnta_v[pl.ds(j * 16, 16)] = offa
        cntb_v[pl.ds(j * 16, 16)] = offb

    zf = jnp.zeros((16,), jnp.float32)

    def zero(i, _):
        accp_v[pl.ds(i * 16, 16)] = zf
        acch_v[pl.ds(i * 16, 16)] = zf
        return 0

    lax.fori_loop(0, CELLS // 16, zero, 0)
    cp1.wait()
    cp2.wait()
    cp3.wait()

    ones = jnp.ones((16,), jnp.int32)
    tmask = jnp.ones((16,), jnp.bool_)

    def stream(cnt_v, lab, cf, ht):
        cbase = plsc.load_gather(cnt_v, [lab])
        occ, _ = plsc.scan_count(lab, mask=tmask)
        rank = cbase + occ.astype(jnp.int32) - 1
        plsc.addupdate_scatter(cnt_v, [lab], ones)
        m = plsc.load_gather(mass_v, [lab])
        r = plsc.load_gather(rcp_v, [lab])
        q0 = (rank.astype(jnp.float32) * r).astype(jnp.int32)
        q0 = jnp.clip(q0, 0, 16)
        q = q0 + (rank >= (q0 + 1) * m).astype(jnp.int32) \
               - (rank < q0 * m).astype(jnp.int32)
        q = jnp.clip(q, 0, 15)
        idx = q * PAD_C + lab
        plsc.addupdate_scatter(accp_v, [idx], cf)
        plsc.addupdate_scatter(acch_v, [idx], ht)

    def body(i, _):
        o = i * 16
        stream(cnta_v, lab_v[pl.ds(o, 16)], cf_v[pl.ds(o, 16)],
               ht_v[pl.ds(o, 16)])
        ob = SCHUNK + o
        stream(cntb_v, lab_v[pl.ds(ob, 16)], cf_v[pl.ds(ob, 16)],
               ht_v[pl.ds(ob, 16)])
        return 0

    lax.fori_loop(0, STEPS, body, 0)
    pltpu.sync_copy(accp_v, accp_out.at[w])
    pltpu.sync_copy(acch_v, acch_out.at[w])


def _final_body(accp_ref, acch_ref, hist_ref, out_ref):
    p = accp_ref[0]
    h = acch_ref[0]
    cnt = hist_ref[0]
    for v in range(1, NW):
        p = p + accp_ref[v]
        h = h + acch_ref[v]
    for v in range(1, NCH):
        cnt = cnt + hist_ref[v]
    mass = _div15(cnt).astype(jnp.float32)
    coef = jnp.float32(1.0) / (mass * jnp.float32(N_BINS * N_CLASS))
    cls = lax.broadcasted_iota(jnp.int32, (1, PAD_C), 1)
    coef = jnp.where(cls < N_CLASS, coef, jnp.float32(0.0))
    d = jnp.abs(p - h) * coef
    b = lax.broadcasted_iota(jnp.int32, (16, PAD_C), 0)
    d = jnp.where(b < N_BINS, d, jnp.float32(0.0))
    out_ref[...] = jnp.sum(d).reshape(1, 1)


def kernel(confidences, hits, labels):
    pad = N_PAD - N
    cf = jnp.concatenate([confidences, jnp.zeros((pad,), jnp.float32)])
    ht = jnp.concatenate([hits, jnp.zeros((pad,), jnp.float32)])
    lab = jnp.concatenate([labels, jnp.full((pad,), PAD_C - 1, jnp.int32)])
    hist = _hist_kernel(lab)
    accp, acch = _main_kernel(cf, ht, lab, hist)
    out = pl.pallas_call(
        _final_body,
        out_shape=jax.ShapeDtypeStruct((1, 1), jnp.float32),
    )(accp.reshape(NW, 16, PAD_C), acch.reshape(NW, 16, PAD_C),
      hist.reshape(NCH, 1, PAD_C))
    return out[0, 0]
